# baseline (speedup 1.0000x reference)
import functools

import jax
import jax.numpy as jnp
from jax import lax
from jax.experimental import pallas as pl
from jax.experimental.pallas import tpu as pltpu
from jax.experimental.pallas import tpu_sc as plsc

_B = 1024
_D = 64
_V = 100000
_BN = 4096


@functools.cache
def _sc_gather(b_per_w, nc):
  mesh = plsc.VectorSubcoreMesh(core_axis_name="c", subcore_axis_name="s")

  @functools.partial(
      pl.kernel,
      mesh=mesh,
      out_type=jax.ShapeDtypeStruct((_B, 2 * _D), jnp.float32),
      scratch_types=[
          pltpu.VMEM((b_per_w,), jnp.int32),
          pltpu.VMEM((b_per_w, 2 * _D), jnp.float32),
          pltpu.SemaphoreType.DMA,
      ],
  )
  def gather(table_hbm, idx_hbm, out_hbm, idx_v, rows_v, sem):
    wid = lax.axis_index("s") * nc + lax.axis_index("c")
    base = wid * b_per_w
    pltpu.sync_copy(idx_hbm.at[pl.ds(base, b_per_w)], idx_v)
    pltpu.async_copy(table_hbm.at[idx_v], rows_v, sem).wait()
    pltpu.sync_copy(rows_v, out_hbm.at[pl.ds(base, b_per_w)])

  return gather


def _mm_body(par_ref, emb2_ref, w_ref, b_ref, out_ref):
  emb = jnp.where(par_ref[...] == 0, emb2_ref[:, :_D], emb2_ref[:, _D:])
  out_ref[...] = (
      jnp.dot(emb, w_ref[...], preferred_element_type=jnp.float32)
      + b_ref[...]
  )


def _matmul(parity, emb2, W, b2d):
  nb = pl.cdiv(_V, _BN)
  return pl.pallas_call(
      _mm_body,
      grid=(nb,),
      in_specs=[
          pl.BlockSpec((_B, 1), lambda i: (0, 0)),
          pl.BlockSpec((_B, 2 * _D), lambda i: (0, 0)),
          pl.BlockSpec((_D, _BN), lambda i: (0, i)),
          pl.BlockSpec((1, _BN), lambda i: (0, i)),
      ],
      out_specs=pl.BlockSpec((_B, _BN), lambda i: (0, i)),
      out_shape=jax.ShapeDtypeStruct((_B, _V), jnp.float32),
  )(parity, emb2, W, b2d)


def kernel(target, emb_table, W, b):
  info = plsc.get_sparse_core_info()
  nw = info.num_cores * info.num_subcores
  tgt = target.astype(jnp.int32)
  emb2 = jnp.take(emb_table.reshape(_V // 2, 2 * _D), tgt >> 1, axis=0)
  parity = (tgt & 1).reshape(_B, 1)
  return _matmul(parity, emb2, W, b.reshape(1, _V))


# speedup vs baseline: 1.0232x; 1.0232x over previous
import functools

import jax
import jax.numpy as jnp
from jax import lax
from jax.experimental import pallas as pl
from jax.experimental.pallas import tpu as pltpu
from jax.experimental.pallas import tpu_sc as plsc

_B = 1024
_D = 64
_V = 100000
_BN = 4096


@functools.cache
def _sc_gather(b_per_w, nc):
  mesh = plsc.VectorSubcoreMesh(core_axis_name="c", subcore_axis_name="s")

  @functools.partial(
      pl.kernel,
      mesh=mesh,
      out_type=jax.ShapeDtypeStruct((_B, _D), jnp.float32),
      scratch_types=[
          pltpu.VMEM((b_per_w,), jnp.int32),
          pltpu.VMEM((b_per_w, _D), jnp.float32),
          pltpu.SemaphoreType.DMA,
      ],
      compiler_params=pltpu.CompilerParams(use_tc_tiling_on_sc=False),
  )
  def gather(table_hbm, idx_hbm, out_hbm, idx_v, rows_v, sem):
    wid = lax.axis_index("s") * nc + lax.axis_index("c")
    base = wid * b_per_w
    pltpu.sync_copy(idx_hbm.at[pl.ds(base, b_per_w)], idx_v)
    pltpu.async_copy(table_hbm.at[idx_v], rows_v, sem).wait()
    pltpu.sync_copy(rows_v, out_hbm.at[pl.ds(base, b_per_w)])

  return gather


def _mm_body(emb_ref, w_ref, b_ref, out_ref):
  out_ref[...] = (
      jnp.dot(emb_ref[...], w_ref[...], preferred_element_type=jnp.float32)
      + b_ref[...]
  )


def _matmul(emb, W, b2d):
  nb = pl.cdiv(_V, _BN)
  return pl.pallas_call(
      _mm_body,
      grid=(nb,),
      in_specs=[
          pl.BlockSpec((_B, _D), lambda i: (0, 0)),
          pl.BlockSpec((_D, _BN), lambda i: (0, i)),
          pl.BlockSpec((1, _BN), lambda i: (0, i)),
      ],
      out_specs=pl.BlockSpec((_B, _BN), lambda i: (0, i)),
      out_shape=jax.ShapeDtypeStruct((_B, _V), jnp.float32),
  )(emb, W, b2d)


def kernel(target, emb_table, W, b):
  info = plsc.get_sparse_core_info()
  nw = info.num_cores * info.num_subcores
  emb = _sc_gather(_B // nw, info.num_cores)(
      emb_table, target.astype(jnp.int32)
  )
  return _matmul(emb, W, b.reshape(1, _V))


# speedup vs baseline: 1.0253x; 1.0020x over previous
import functools

import jax
import jax.numpy as jnp
from jax import lax
from jax.experimental import pallas as pl
from jax.experimental.pallas import tpu as pltpu
from jax.experimental.pallas import tpu_sc as plsc

_B = 1024
_D = 64
_V = 100000
_BN = 4096


@functools.cache
def _sc_gather(b_per_w, nc):
  mesh = plsc.VectorSubcoreMesh(core_axis_name="c", subcore_axis_name="s")

  @functools.partial(
      pl.kernel,
      mesh=mesh,
      out_type=jax.ShapeDtypeStruct((_B, _D), jnp.float32),
      scratch_types=[
          pltpu.VMEM((b_per_w,), jnp.int32),
          pltpu.VMEM((b_per_w, _D), jnp.float32),
          pltpu.SemaphoreType.DMA,
      ],
      compiler_params=pltpu.CompilerParams(
          use_tc_tiling_on_sc=False, skip_device_barrier=True
      ),
  )
  def gather(table_hbm, idx_hbm, out_hbm, idx_v, rows_v, sem):
    wid = lax.axis_index("s") * nc + lax.axis_index("c")
    base = wid * b_per_w
    pltpu.sync_copy(idx_hbm.at[pl.ds(base, b_per_w)], idx_v)
    pltpu.async_copy(table_hbm.at[idx_v], rows_v, sem).wait()
    pltpu.sync_copy(rows_v, out_hbm.at[pl.ds(base, b_per_w)])

  return gather


def _mm_body(emb_ref, w_ref, b_ref, out_ref):
  out_ref[...] = (
      jnp.dot(emb_ref[...], w_ref[...], preferred_element_type=jnp.float32)
      + b_ref[...]
  )


def _matmul(emb, W, b2d):
  nb = pl.cdiv(_V, _BN)
  return pl.pallas_call(
      _mm_body,
      grid=(nb,),
      in_specs=[
          pl.BlockSpec((_B, _D), lambda i: (0, 0)),
          pl.BlockSpec((_D, _BN), lambda i: (0, i)),
          pl.BlockSpec((1, _BN), lambda i: (0, i)),
      ],
      out_specs=pl.BlockSpec((_B, _BN), lambda i: (0, i)),
      out_shape=jax.ShapeDtypeStruct((_B, _V), jnp.float32),
  )(emb, W, b2d)


def kernel(target, emb_table, W, b):
  info = plsc.get_sparse_core_info()
  nw = info.num_cores * info.num_subcores
  emb = _sc_gather(_B // nw, info.num_cores)(
      emb_table, target.astype(jnp.int32)
  )
  return _matmul(emb, W, b.reshape(1, _V))
